# baseline (speedup 1.0000x reference)
import functools

import jax
import jax.numpy as jnp
from jax import lax
from jax.experimental import pallas as pl
from jax.experimental.pallas import tpu as pltpu, tpu_sc as plsc

_NUM_FIELDS = 26
_VOCAB = 100000
_EMBED_DIM = 16
_BATCH = 4096

_info = plsc.get_sparse_core_info()
_NC, _NS, _L = _info.num_cores, _info.num_subcores, _info.num_lanes
_NW = _NC * _NS
_BPW = _BATCH // _NW
_NCHUNK = _BPW // _L


def _body(idx_hbm, emb_hbm, fo_hbm, out_emb, out_fo,
          idx2d, lin_idx, emb_buf, fo_buf, acc, sem):
    wid = lax.axis_index("s") * _NC + lax.axis_index("c")
    base = wid * _BPW

    pltpu.sync_copy(idx_hbm.at[:, pl.ds(base, _BPW)], idx2d)

    def build(f, carry):
        off = f * _VOCAB
        for c in range(_NCHUNK):
            lin_idx[f, pl.ds(c * _L, _L)] = idx2d[f, pl.ds(c * _L, _L)] + off
        return carry

    lax.fori_loop(0, _NUM_FIELDS, build, 0)

    def gather_emb(f, carry):
        pltpu.async_copy(emb_hbm.at[lin_idx.at[f]], emb_buf.at[f], sem).wait()
        pltpu.sync_copy(emb_buf.at[f], out_emb.at[pl.ds(base, _BPW), f])
        return carry

    lax.fori_loop(0, _NUM_FIELDS, gather_emb, 0)

    def gather_fo(f, carry):
        pltpu.async_copy(fo_hbm.at[lin_idx.at[f]], fo_buf.at[f], sem).wait()
        return carry

    lax.fori_loop(0, _NUM_FIELDS, gather_fo, 0)

    for c in range(_NCHUNK):
        acc[pl.ds(c * _L, _L)] = jnp.zeros((_L,), jnp.float32)

    def reduce_fo(f, carry):
        for c in range(_NCHUNK):
            plsc.addupdate(acc.at[pl.ds(c * _L, _L)],
                           fo_buf[f, pl.ds(c * _L, _L)])
        return carry

    lax.fori_loop(0, _NUM_FIELDS, reduce_fo, 0)

    pltpu.sync_copy(acc, out_fo.at[pl.ds(base, _BPW)])


_fe_kernel = functools.partial(
    pl.kernel,
    out_type=[
        jax.ShapeDtypeStruct((_BATCH, _NUM_FIELDS, _EMBED_DIM), jnp.float32),
        jax.ShapeDtypeStruct((_BATCH,), jnp.float32),
    ],
    mesh=plsc.VectorSubcoreMesh(core_axis_name="c", subcore_axis_name="s"),
    compiler_params=pltpu.CompilerParams(use_tc_tiling_on_sc=False),
    scratch_types=[
        pltpu.VMEM((_NUM_FIELDS, _BPW), jnp.int32),
        pltpu.VMEM((_NUM_FIELDS, _BPW), jnp.int32),
        pltpu.VMEM((_NUM_FIELDS, _BPW, _EMBED_DIM), jnp.float32),
        pltpu.VMEM((_NUM_FIELDS, _BPW), jnp.float32),
        pltpu.VMEM((_BPW,), jnp.float32),
        pltpu.SemaphoreType.DMA,
    ],
)(_body)


def kernel(indices, emb_tables, fo_tables):
    idx = indices.astype(jnp.int32)
    emb_flat = emb_tables.reshape(_NUM_FIELDS * _VOCAB, _EMBED_DIM)
    fo_flat = fo_tables.reshape(_NUM_FIELDS * _VOCAB)
    out_emb, out_fo = _fe_kernel(idx, emb_flat, fo_flat)
    first_order = out_fo.reshape(_BATCH, 1)
    field_embeddings = out_emb
    flat_embeddings = out_emb.reshape(_BATCH, _NUM_FIELDS * _EMBED_DIM)
    return (first_order, field_embeddings, flat_embeddings)
